# baseline (speedup 1.0000x reference)
import numpy as np

import jax
import jax.numpy as jnp
from jax.experimental import pallas as pl
from jax.experimental.pallas import tpu as pltpu

_B = 4
_N = 512
_T = 100
_SPEED = 0.05
_LANES = 128
_ROWS_PER_BATCH = (_N * _N) // _LANES
_RPI = 64
_TOTAL = _B * _N * _N
_LN_NOTFLIP = float(np.log(np.float32(1.0 - 2.0 * _SPEED)))


def _threefry_bits(p):
    ks0 = jnp.uint32(0)
    ks1 = jnp.uint32(42)
    ks2 = jnp.uint32(42 ^ 0x1BD11BDA)
    ksl = (ks0, ks1, ks2)
    rot = ((13, 15, 26, 6), (17, 29, 16, 24))
    x0 = jnp.zeros_like(p)
    x1 = p + ks1
    for i in range(5):
        for r in rot[i % 2]:
            x0 = x0 + x1
            x1 = (x1 << jnp.uint32(r)) | (x1 >> jnp.uint32(32 - r))
            x1 = x1 ^ x0
        x0 = x0 + ksl[(i + 1) % 3]
        x1 = x1 + ksl[(i + 2) % 3] + jnp.uint32(i + 1)
    return x0 ^ x1


def _uniform(bits):
    return jax.lax.bitcast_convert_type(
        (bits >> jnp.uint32(9)) | jnp.uint32(0x3F800000), jnp.float32
    ) - jnp.float32(1.0)


def _flip(ts):
    return jnp.float32(0.5) * (jnp.float32(1.0) - jnp.exp(ts * jnp.float32(_LN_NOTFLIP)))


def _kern(t_ref, b2_ref, adj_ref, w1_ref, b1_ref, w2_ref, out_ref):
    one = jnp.float32(1.0)
    total = jnp.zeros((1, _LANES), jnp.float32)
    for b in range(_B):
        tv = jnp.full((1, _LANES), t_ref[b].astype(jnp.float32))
        te = tv + one
        tp = jnp.where(tv == 0.0, jnp.float32(_T), tv)
        fe = _flip(te)
        fp = _flip(tp)
        f1 = _flip(jnp.full_like(tv, 1.0))
        tn = tv * jnp.float32(1.0 / _T)

        w1a = w1_ref[pl.ds(0, 1), :]
        w1b = w1_ref[pl.ds(1, 1), :]
        b1v = b1_ref[...]
        w2v = w2_ref[...]
        b2v = b2_ref[0]
        h0 = jnp.maximum(tn * w1b + b1v, 0.0)
        h1 = jnp.maximum(w1a + tn * w1b + b1v, 0.0)
        z0 = jnp.sum(h0 * w2v, axis=1, keepdims=True) + b2v
        z1 = jnp.sum(h1 * w2v, axis=1, keepdims=True) + b2v
        eps = jnp.float32(1e-7)
        p0 = jnp.clip(jax.nn.sigmoid(z0), eps, one - eps)
        p1 = jnp.clip(jax.nn.sigmoid(z1), eps, one - eps)
        lp0, lq0 = jnp.log(p0), jnp.log1p(-p0)
        lp1, lq1 = jnp.log(p1), jnp.log1p(-p1)

        ws = []
        for s in range(2):
            prior = (one - fp) if s == 0 else fp
            for y in range(2):
                lik = (one - f1) if y == 0 else f1
                evid = (one - fe) if s == y else fe
                tgt = jnp.clip(lik * prior / evid, 0.0, 1.0)
                lp, lq = (lp0, lq0) if y == 0 else (lp1, lq1)
                ws.append(-(tgt * lp + (one - tgt) * lq))
        w00, w01, w10, w11 = ws

        def cipher_tile(i):
            r0 = b * _ROWS_PER_BATCH + i * _RPI
            e = (
                (jax.lax.broadcasted_iota(jnp.int32, (_RPI, _LANES), 0) + r0) * _LANES
                + jax.lax.broadcasted_iota(jnp.int32, (_RPI, _LANES), 1)
            )
            pc = (2 * e).astype(jnp.uint32)
            u0 = _uniform(_threefry_bits(pc))
            u1 = _uniform(_threefry_bits(pc + jnp.uint32(1)))
            return u0, u1

        def finish_tile(i, u0, u1):
            r0 = b * _ROWS_PER_BATCH + i * _RPI
            s = adj_ref[pl.ds(r0, _RPI), :]
            g0 = jnp.log2(u0)
            g1 = jnp.log2(u1)
            is0 = s == 0
            q0 = jnp.where(is0, one - fe, fe)
            q1 = jnp.where(is0, fe, one - fe)
            y = (q0 * g1) > (q1 * g0)
            val = jnp.where(is0, jnp.where(y, w01, w00), jnp.where(y, w11, w10))
            return jnp.sum(val, axis=0, keepdims=True)

        n_iter = _ROWS_PER_BATCH // _RPI

        def body(i, carry):
            acc, u0, u1 = carry
            nxt = cipher_tile(i + 1)
            acc = acc + finish_tile(i, u0, u1)
            return (acc, *nxt)

        u00, u10 = cipher_tile(0)
        acc, u0_l, u1_l = jax.lax.fori_loop(
            0, n_iter - 1, body, (jnp.zeros((1, _LANES), jnp.float32), u00, u10)
        )
        total = total + acc + finish_tile(n_iter - 1, u0_l, u1_l)
    out_ref[...] = jnp.sum(total, axis=1, keepdims=True) * jnp.float32(1.0 / _TOTAL)


def kernel(adj_start, t, W1, b1, W2, b2):
    adj2 = adj_start.reshape(_B * _ROWS_PER_BATCH, _LANES)
    b1r = b1.reshape(1, _LANES)
    w2r = W2.reshape(1, _LANES)
    out = pl.pallas_call(
        _kern,
        out_shape=jax.ShapeDtypeStruct((1, 1), jnp.float32),
        in_specs=[
            pl.BlockSpec(memory_space=pltpu.SMEM),
            pl.BlockSpec(memory_space=pltpu.SMEM),
            pl.BlockSpec(memory_space=pltpu.VMEM),
            pl.BlockSpec(memory_space=pltpu.VMEM),
            pl.BlockSpec(memory_space=pltpu.VMEM),
            pl.BlockSpec(memory_space=pltpu.VMEM),
        ],
    )(t, b2, adj2, W1, b1r, w2r)
    return out[0, 0]


# speedup vs baseline: 1.0036x; 1.0036x over previous
import numpy as np

import jax
import jax.numpy as jnp
from jax.experimental import pallas as pl
from jax.experimental.pallas import tpu as pltpu

_B = 4
_N = 512
_T = 100
_SPEED = 0.05
_LANES = 128
_ROWS_PER_BATCH = (_N * _N) // _LANES
_RPI = 64
_TOTAL = _B * _N * _N
_LN_NOTFLIP = float(np.log(np.float32(1.0 - 2.0 * _SPEED)))


def _threefry_bits(p):
    ks0 = jnp.uint32(0)
    ks1 = jnp.uint32(42)
    ks2 = jnp.uint32(42 ^ 0x1BD11BDA)
    ksl = (ks0, ks1, ks2)
    rot = ((13, 15, 26, 6), (17, 29, 16, 24))
    x0 = jnp.zeros_like(p)
    x1 = p + ks1
    for i in range(5):
        for r in rot[i % 2]:
            x0 = x0 + x1
            x1 = (x1 << jnp.uint32(r)) | (x1 >> jnp.uint32(32 - r))
            x1 = x1 ^ x0
        x0 = x0 + ksl[(i + 1) % 3]
        x1 = x1 + ksl[(i + 2) % 3] + jnp.uint32(i + 1)
    return x0 ^ x1


def _uniform(bits):
    return jax.lax.bitcast_convert_type(
        (bits >> jnp.uint32(9)) | jnp.uint32(0x3F800000), jnp.float32
    ) - jnp.float32(1.0)


def _flip(ts):
    return jnp.float32(0.5) * (jnp.float32(1.0) - jnp.exp(ts * jnp.float32(_LN_NOTFLIP)))


def _kern(t_ref, b2_ref, adj_ref, w1_ref, b1_ref, w2_ref, out_ref):
    one = jnp.float32(1.0)
    total = jnp.zeros((1, _LANES), jnp.float32)

    tv = jnp.concatenate(
        [jnp.full((1, _LANES), t_ref[i].astype(jnp.float32)) for i in range(_B)],
        axis=0,
    )
    te = tv + one
    tp = jnp.where(tv == 0.0, jnp.float32(_T), tv)
    fe_all = _flip(te)
    fp = _flip(tp)
    f1 = _flip(jnp.full((1, _LANES), 1.0, jnp.float32))
    tn = tv * jnp.float32(1.0 / _T)

    w1a = w1_ref[pl.ds(0, 1), :]
    w1b = w1_ref[pl.ds(1, 1), :]
    b1v = b1_ref[...]
    w2v = w2_ref[...]
    b2v = b2_ref[0]
    h0 = jnp.maximum(tn * w1b + b1v, 0.0)
    h1 = jnp.maximum(w1a + tn * w1b + b1v, 0.0)
    z0 = jnp.sum(h0 * w2v, axis=1, keepdims=True) + b2v
    z1 = jnp.sum(h1 * w2v, axis=1, keepdims=True) + b2v
    eps = jnp.float32(1e-7)
    p0 = jnp.clip(jax.nn.sigmoid(z0), eps, one - eps)
    p1 = jnp.clip(jax.nn.sigmoid(z1), eps, one - eps)
    lp0, lq0 = jnp.log(p0), jnp.log1p(-p0)
    lp1, lq1 = jnp.log(p1), jnp.log1p(-p1)

    ws_all = []
    for s in range(2):
        prior = (one - fp) if s == 0 else fp
        for y in range(2):
            lik = (one - f1) if y == 0 else f1
            evid = (one - fe_all) if s == y else fe_all
            tgt = jnp.clip(lik * prior / evid, 0.0, 1.0)
            lp, lq = (lp0, lq0) if y == 0 else (lp1, lq1)
            ws_all.append(-(tgt * lp + (one - tgt) * lq))

    for b in range(_B):
        fe = fe_all[b : b + 1, :]
        w00, w01, w10, w11 = (w[b : b + 1, :] for w in ws_all)

        def cipher_tile(i):
            r0 = b * _ROWS_PER_BATCH + i * _RPI
            e = (
                (jax.lax.broadcasted_iota(jnp.int32, (_RPI, _LANES), 0) + r0) * _LANES
                + jax.lax.broadcasted_iota(jnp.int32, (_RPI, _LANES), 1)
            )
            pc = (2 * e).astype(jnp.uint32)
            u0 = _uniform(_threefry_bits(pc))
            u1 = _uniform(_threefry_bits(pc + jnp.uint32(1)))
            return u0, u1

        def finish_tile(i, u0, u1):
            r0 = b * _ROWS_PER_BATCH + i * _RPI
            s = adj_ref[pl.ds(r0, _RPI), :]
            g0 = jnp.log2(u0)
            g1 = jnp.log2(u1)
            is0 = s == 0
            q0 = jnp.where(is0, one - fe, fe)
            q1 = jnp.where(is0, fe, one - fe)
            y = (q0 * g1) > (q1 * g0)
            val = jnp.where(is0, jnp.where(y, w01, w00), jnp.where(y, w11, w10))
            return jnp.sum(val, axis=0, keepdims=True)

        n_iter = _ROWS_PER_BATCH // _RPI

        def body(i, carry):
            acc, u0, u1 = carry
            nxt = cipher_tile(i + 1)
            acc = acc + finish_tile(i, u0, u1)
            return (acc, *nxt)

        u00, u10 = cipher_tile(0)
        acc, u0_l, u1_l = jax.lax.fori_loop(
            0, n_iter - 1, body, (jnp.zeros((1, _LANES), jnp.float32), u00, u10)
        )
        total = total + acc + finish_tile(n_iter - 1, u0_l, u1_l)
    out_ref[...] = jnp.sum(total, axis=1, keepdims=True) * jnp.float32(1.0 / _TOTAL)


def kernel(adj_start, t, W1, b1, W2, b2):
    adj2 = adj_start.reshape(_B * _ROWS_PER_BATCH, _LANES)
    b1r = b1.reshape(1, _LANES)
    w2r = W2.reshape(1, _LANES)
    out = pl.pallas_call(
        _kern,
        out_shape=jax.ShapeDtypeStruct((1, 1), jnp.float32),
        in_specs=[
            pl.BlockSpec(memory_space=pltpu.SMEM),
            pl.BlockSpec(memory_space=pltpu.SMEM),
            pl.BlockSpec(memory_space=pltpu.VMEM),
            pl.BlockSpec(memory_space=pltpu.VMEM),
            pl.BlockSpec(memory_space=pltpu.VMEM),
            pl.BlockSpec(memory_space=pltpu.VMEM),
        ],
    )(t, b2, adj2, W1, b1r, w2r)
    return out[0, 0]
